# baseline (speedup 1.0000x reference)
import functools

import jax
import jax.numpy as jnp
from jax import lax
from jax.experimental import pallas as pl
from jax.experimental.pallas import tpu as pltpu
from jax.experimental.pallas import tpu_sc as plsc

EMBED = 32
GRP = 128


@functools.lru_cache(maxsize=None)
def _make_gather(n_idx, vocab, embed):
    info = plsc.get_sparse_core_info()
    nc, ns = info.num_cores, info.num_subcores
    nw = nc * ns
    b_per_w = n_idx // nw
    assert b_per_w * nw == n_idx and b_per_w % GRP == 0
    n_grp = b_per_w // GRP
    mesh = plsc.VectorSubcoreMesh(core_axis_name="c", subcore_axis_name="s")

    @functools.partial(
        pl.kernel,
        mesh=mesh,
        compiler_params=pltpu.CompilerParams(use_tc_tiling_on_sc=False),
        out_type=jax.ShapeDtypeStruct((n_idx, embed), jnp.float32),
        scratch_types=[
            pltpu.VMEM((n_grp, GRP), jnp.int32),
            pltpu.VMEM((GRP, embed), jnp.float32),
            pltpu.SemaphoreType.DMA,
        ],
    )
    def k(idx_hbm, table_hbm, out_hbm, idx_v, rows_v, sem):
        wid = lax.axis_index("s") * nc + lax.axis_index("c")
        base = wid * b_per_w
        pltpu.sync_copy(idx_hbm.at[wid], idx_v)

        def body(j, carry):
            pltpu.async_copy(table_hbm.at[idx_v.at[j]], rows_v, sem).wait()
            pltpu.sync_copy(rows_v, out_hbm.at[pl.ds(base + j * GRP, GRP)])
            return carry

        lax.fori_loop(0, n_grp, body, 0)

    return k


def kernel(inputs, embeddings):
    b, l = inputs.shape
    n = b * l
    info = plsc.get_sparse_core_info()
    nw = info.num_cores * info.num_subcores
    idx3 = inputs.reshape(nw, n // nw // GRP, GRP).astype(jnp.int32)
    vocab, embed = embeddings.shape
    out = _make_gather(n, vocab, embed)(idx3, embeddings)
    return out.reshape(b, l, embed)


# speedup vs baseline: 1.0783x; 1.0783x over previous
import functools

import jax
import jax.numpy as jnp
from jax import lax
from jax.experimental import pallas as pl
from jax.experimental.pallas import tpu as pltpu
from jax.experimental.pallas import tpu_sc as plsc

EMBED = 32
CHUNK = 1280
NBUF = 3


@functools.lru_cache(maxsize=None)
def _make_gather(n_idx, vocab, embed):
    info = plsc.get_sparse_core_info()
    nc, ns = info.num_cores, info.num_subcores
    nw = nc * ns
    b_per_w = n_idx // nw
    n_chunk = b_per_w // CHUNK
    assert n_chunk * CHUNK * nw == n_idx
    mesh = plsc.VectorSubcoreMesh(core_axis_name="c", subcore_axis_name="s")

    @functools.partial(
        pl.kernel,
        mesh=mesh,
        compiler_params=pltpu.CompilerParams(use_tc_tiling_on_sc=False),
        out_type=jax.ShapeDtypeStruct((nw * n_chunk, CHUNK, embed), jnp.float32),
        scratch_types=[
            pltpu.VMEM((n_chunk, CHUNK), jnp.int32),
            pltpu.VMEM((NBUF, CHUNK, embed), jnp.float32),
            pltpu.SemaphoreType.DMA((NBUF,)),
            pltpu.SemaphoreType.DMA((NBUF,)),
        ],
    )
    def k(idx_hbm, table_hbm, out_hbm, idx_v, rows_v, gsem, osem):
        wid = lax.axis_index("s") * nc + lax.axis_index("c")
        pltpu.sync_copy(idx_hbm.at[wid], idx_v)

        def fire(c):
            b = c % NBUF
            pltpu.async_copy(table_hbm.at[idx_v.at[c]], rows_v.at[b], gsem.at[b])

        for c in range(min(NBUF, n_chunk)):
            fire(c)
        for c in range(n_chunk):
            b = c % NBUF
            pltpu.make_async_copy(
                table_hbm.at[idx_v.at[c]], rows_v.at[b], gsem.at[b]
            ).wait()
            pltpu.async_copy(rows_v.at[b], out_hbm.at[wid * n_chunk + c], osem.at[b])
            if c + NBUF < n_chunk:
                pltpu.make_async_copy(
                    rows_v.at[b], out_hbm.at[wid * n_chunk + c], osem.at[b]
                ).wait()
                fire(c + NBUF)
        for c in range(max(0, n_chunk - NBUF), n_chunk):
            b = c % NBUF
            pltpu.make_async_copy(
                rows_v.at[b], out_hbm.at[wid * n_chunk + c], osem.at[b]
            ).wait()

    return k


def kernel(inputs, embeddings):
    b, l = inputs.shape
    n = b * l
    info = plsc.get_sparse_core_info()
    nw = info.num_cores * info.num_subcores
    n_chunk = n // nw // CHUNK
    idx4 = inputs.reshape(nw, n_chunk, CHUNK).astype(jnp.int32)
    vocab, embed = embeddings.shape
    out = _make_gather(n, vocab, embed)(idx4, embeddings)
    return out.reshape(b, l, embed)
